# baseline (speedup 1.0000x reference)
import functools

import jax
import jax.numpy as jnp
from jax import lax
from jax.experimental import pallas as pl
from jax.experimental.pallas import tpu as pltpu
from jax.experimental.pallas import tpu_sc as plsc

NUM_NT = 1024
NT_DIM = 64
DECAY = 0.99

_R = 1024
_W = 128
_NTILE = 16
_RPT = 800
_CHUNK = 160
_SUB = 80


def _tc_body(xt_ref, mask_ref, cb_ref, cbt_ref,
             idx_ref, push_ref, w_ref, q_ref):
    cb = cb_ref[...]
    cbt = cbt_ref[...]
    cbsq = jnp.sum(cb * cb, axis=1, keepdims=True)
    sel0 = (lax.broadcasted_iota(jnp.int32, (_W - NT_DIM, 128), 0) == 0
            ).astype(jnp.float32)
    kiota = lax.broadcasted_iota(jnp.int32, (NUM_NT, 128), 0)
    idx_rows = []
    for p in range(8):
        xT = xt_ref[p]
        scoresT = lax.dot_general(cb, xT, (((1,), (0,)), ((), ())),
                                  preferred_element_type=jnp.float32)
        xsqT = jnp.sum(xT * xT, axis=0, keepdims=True)
        d2T = xsqT - 2.0 * scoresT + cbsq
        idx_p = jnp.argmin(d2T, axis=0).astype(jnp.int32)
        idx_rows.append(idx_p[None, :])
        onehotT = (kiota == idx_p[None, :]).astype(jnp.float32)
        q_ref[p] = lax.dot_general(cbt, onehotT, (((1,), (0,)), ((), ())),
                                   preferred_element_type=jnp.float32)
        mrow = mask_ref[0, p, :][None, :]
        wT = jnp.concatenate([xT * mrow, sel0 * mrow], axis=0)
        w_ref[pl.ds(p * 128, 128), :] = jnp.transpose(wT, (1, 0))
    idx2 = jnp.concatenate(idx_rows, axis=0)
    m2 = mask_ref[0]
    idx_ref[0] = idx2
    push_ref[0] = idx2 * m2.astype(jnp.int32)


def _ema_body(p_ref, maccx_ref, newx_ref):
    newx_ref[...] = (DECAY * maccx_ref[...]
                     + (1.0 - DECAY) * (p_ref[0] + p_ref[1]))


def _sc_body(idx_hbm, w_hbm, zero_hbm,
             part_hbm,
             idx_v, wbuf0, wbuf1, accum_sh, sem_g, sem_w):
    cid = lax.axis_index("c")
    sid = lax.axis_index("s")
    rows_acc = NUM_NT // _NTILE
    wid = cid * _NTILE + sid
    base = wid * _RPT
    nch = _RPT // _CHUNK
    nsub = _CHUNK // _SUB
    wbufs = (wbuf0, wbuf1)

    idx_copies = [
        pltpu.async_copy(idx_hbm.at[pl.ds(base + j * _SUB, _SUB)],
                         idx_v.at[j], sem_g)
        for j in range(_RPT // _SUB)
    ]
    r0 = sid * rows_acc
    pltpu.sync_copy(zero_hbm.at[pl.ds(r0, rows_acc)],
                    accum_sh.at[pl.ds(r0, rows_acc)])
    wdma = {0: pltpu.async_copy(w_hbm.at[wid, pl.ds(0, _CHUNK)],
                                wbufs[0], sem_w)}
    for c in idx_copies:
        c.wait()
    plsc.subcore_barrier()
    for ch in range(nch):
        wdma[ch].wait()
        if ch + 1 < nch:
            wdma[ch + 1] = pltpu.async_copy(
                w_hbm.at[wid, pl.ds((ch + 1) * _CHUNK, _CHUNK)],
                wbufs[(ch + 1) & 1], sem_w)
        for j in range(nsub):
            pltpu.sync_copy(
                wbufs[ch & 1].at[pl.ds(j * _SUB, _SUB)],
                accum_sh.at[idx_v.at[ch * nsub + j]],
                add=True)
    plsc.subcore_barrier()
    pltpu.sync_copy(accum_sh.at[pl.ds(r0, rows_acc)],
                    part_hbm.at[cid, pl.ds(r0, rows_acc)])


def kernel(codes, step_mask, codebook, m_acc, n_acc):
    B, T, two, D = codes.shape
    K = codebook.shape[0]
    N = B * T * two
    nsteps = N // _R
    xt3 = codes.transpose(1, 2, 3, 0).reshape(T * two, D, B)
    maskf = jnp.broadcast_to(
        step_mask.T[:, None, :], (T, two, B)).reshape(nsteps, 8, 128)

    idx3, push3, weighted, quantT = pl.pallas_call(
        _tc_body,
        grid=(nsteps,),
        in_specs=[
            pl.BlockSpec((8, D, 128), lambda i: (i, 0, 0)),
            pl.BlockSpec((1, 8, 128), lambda i: (i, 0, 0)),
            pl.BlockSpec((K, D), lambda i: (0, 0)),
            pl.BlockSpec((D, K), lambda i: (0, 0)),
        ],
        out_specs=[
            pl.BlockSpec((1, 8, 128), lambda i: (i, 0, 0)),
            pl.BlockSpec((1, 8, 128), lambda i: (i, 0, 0)),
            pl.BlockSpec((_R, _W), lambda i: (i, 0)),
            pl.BlockSpec((8, D, 128), lambda i: (i, 0, 0)),
        ],
        out_shape=(
            jax.ShapeDtypeStruct((nsteps, 8, 128), jnp.int32),
            jax.ShapeDtypeStruct((nsteps, 8, 128), jnp.int32),
            jax.ShapeDtypeStruct((N, _W), jnp.float32),
            jax.ShapeDtypeStruct((T * two, D, B), jnp.float32),
        ),
    )(xt3, maskf, codebook, codebook.T)

    idx_t = idx3.reshape(N)
    w_t = weighted.reshape(2 * _NTILE, _RPT, _W)
    maccx = jnp.concatenate(
        [m_acc, n_acc[:, None], jnp.zeros((K, _W - D - 1), jnp.float32)],
        axis=1)
    zeros = jnp.zeros((K, _W), jnp.float32)

    mesh = plsc.VectorSubcoreMesh(core_axis_name="c", subcore_axis_name="s")
    sc = functools.partial(
        pl.kernel,
        mesh=mesh,
        out_type=jax.ShapeDtypeStruct((2, K, _W), jnp.float32),
        scratch_types=[
            pltpu.VMEM((_RPT // _SUB, _SUB), jnp.int32),
            pltpu.VMEM((_CHUNK, _W), jnp.float32),
            pltpu.VMEM((_CHUNK, _W), jnp.float32),
            pltpu.VMEM_SHARED((K, _W), jnp.float32),
            pltpu.SemaphoreType.DMA,
            pltpu.SemaphoreType.DMA,
        ],
    )(_sc_body)
    parts = sc(idx_t, w_t, zeros)

    newx = pl.pallas_call(
        _ema_body,
        in_specs=[
            pl.BlockSpec((2, K, _W), lambda: (0, 0, 0)),
            pl.BlockSpec((K, _W), lambda: (0, 0)),
        ],
        out_specs=pl.BlockSpec((K, _W), lambda: (0, 0)),
        out_shape=jax.ShapeDtypeStruct((K, _W), jnp.float32),
    )(parts, maccx)

    return (quantT.reshape(T, two, D, B).transpose(3, 0, 1, 2),
            push3.reshape(T, two, B).transpose(2, 0, 1),
            newx[:, :D],
            newx[:, D])


# speedup vs baseline: 1.2703x; 1.2703x over previous
import functools

import jax
import jax.numpy as jnp
from jax import lax
from jax.experimental import pallas as pl
from jax.experimental.pallas import tpu as pltpu
from jax.experimental.pallas import tpu_sc as plsc

NUM_NT = 1024
NT_DIM = 64
DECAY = 0.99

_R = 5120
_P = _R // 128
_W = 128
_NTILE = 16
_RPT = 1600
_CHUNK = 160
_SUB = 80


def _tc_body(xt_ref, mask_ref, cb_ref,
             idx_ref, push_ref, w_ref):
    cb = cb_ref[...]
    cbsq = jnp.sum(cb * cb, axis=1, keepdims=True)
    sel0 = (lax.broadcasted_iota(jnp.int32, (_W - NT_DIM, 128), 0) == 0
            ).astype(jnp.float32)
    idx_rows = []
    for p in range(_P):
        xT = xt_ref[p]
        scoresT = lax.dot_general(cb, xT, (((1,), (0,)), ((), ())),
                                  preferred_element_type=jnp.float32)
        xsqT = jnp.sum(xT * xT, axis=0, keepdims=True)
        d2T = xsqT - 2.0 * scoresT + cbsq
        idx_rows.append(jnp.argmin(d2T, axis=0).astype(jnp.int32)[None, :])
        mrow = mask_ref[0, p, :][None, :]
        wT = jnp.concatenate([xT * mrow, sel0 * mrow], axis=0)
        w_ref[pl.ds(p * 128, 128), :] = jnp.transpose(wT, (1, 0))
    idx2 = jnp.concatenate(idx_rows, axis=0)
    m2 = mask_ref[0]
    idx_ref[0] = idx2
    push_ref[0] = idx2 * m2.astype(jnp.int32)


def _sc_body(idx_hbm, w_hbm, cb_hbm, maccx_hbm, zero_hbm,
             quant_hbm, newx_hbm,
             idx_g, idx_v, gbuf0, gbuf1, wbuf0, wbuf1, eacc_v, emax_v,
             accum_sh, sem_g, sem_o, sem_w):
    cid = lax.axis_index("c")
    sid = lax.axis_index("s")
    rows_acc = NUM_NT // _NTILE
    base = sid * _RPT
    nch = _RPT // _CHUNK
    nsub = _CHUNK // _SUB
    gbufs = (gbuf0, gbuf1)
    wbufs = (wbuf0, wbuf1)

    @pl.when(cid == 1)
    def _gather():
        pltpu.sync_copy(idx_hbm.at[pl.ds(base, _RPT)], idx_g)
        gh, oh = {}, {}
        for ch in range(nch + 1):
            if ch >= 1:
                for c in gh[ch - 1]:
                    c.wait()
                oh[ch - 1] = pltpu.async_copy(
                    gbufs[(ch - 1) & 1],
                    quant_hbm.at[pl.ds(base + (ch - 1) * _CHUNK, _CHUNK)],
                    sem_o)
            if ch >= 2:
                oh[ch - 2].wait()
            if ch < nch:
                gh[ch] = [
                    pltpu.async_copy(
                        cb_hbm.at[idx_g.at[pl.ds(ch * _CHUNK + j * _SUB,
                                                 _SUB)]],
                        gbufs[ch & 1].at[pl.ds(j * _SUB, _SUB)], sem_g)
                    for j in range(nsub)
                ]
        oh[nch - 1].wait()

    @pl.when(cid == 0)
    def _scatter():
        idx_copies = [
            pltpu.async_copy(idx_hbm.at[pl.ds(base + j * _SUB, _SUB)],
                             idx_v.at[j], sem_g)
            for j in range(_RPT // _SUB)
        ]
        r0 = sid * rows_acc
        pltpu.sync_copy(zero_hbm.at[pl.ds(r0, rows_acc)],
                        accum_sh.at[pl.ds(r0, rows_acc)])
        wdma = {0: pltpu.async_copy(w_hbm.at[sid, pl.ds(0, _CHUNK)],
                                    wbufs[0], sem_w)}
        for c in idx_copies:
            c.wait()
        plsc.subcore_barrier()
        for ch in range(nch):
            wdma[ch].wait()
            if ch + 1 < nch:
                wdma[ch + 1] = pltpu.async_copy(
                    w_hbm.at[sid, pl.ds((ch + 1) * _CHUNK, _CHUNK)],
                    wbufs[(ch + 1) & 1], sem_w)
            for j in range(nsub):
                pltpu.sync_copy(
                    wbufs[ch & 1].at[pl.ds(j * _SUB, _SUB)],
                    accum_sh.at[idx_v.at[ch * nsub + j]],
                    add=True)
        plsc.subcore_barrier()
        pltpu.sync_copy(accum_sh.at[pl.ds(r0, rows_acc)], eacc_v)
        pltpu.sync_copy(maccx_hbm.at[pl.ds(r0, rows_acc)], emax_v)

        def row_fn(rr, _):
            for c4 in range(_W // 16):
                s = pl.ds(c4 * 16, 16)
                emax_v[rr, s] = (DECAY * emax_v[rr, s]
                                 + (1.0 - DECAY) * eacc_v[rr, s])
            return 0

        lax.fori_loop(0, rows_acc, row_fn, 0)
        pltpu.sync_copy(emax_v, newx_hbm.at[pl.ds(r0, rows_acc)])


def kernel(codes, step_mask, codebook, m_acc, n_acc):
    B, T, two, D = codes.shape
    K = codebook.shape[0]
    N = B * T * two
    nsteps = N // _R
    xt3 = codes.transpose(1, 2, 3, 0).reshape(T * two, D, B)
    maskf = jnp.broadcast_to(
        step_mask.T[:, None, :], (T, two, B)).reshape(nsteps, _P, 128)

    idx3, push3, weighted = pl.pallas_call(
        _tc_body,
        grid=(nsteps,),
        in_specs=[
            pl.BlockSpec((_P, D, 128), lambda i: (i, 0, 0)),
            pl.BlockSpec((1, _P, 128), lambda i: (i, 0, 0)),
            pl.BlockSpec((K, D), lambda i: (0, 0)),
        ],
        out_specs=[
            pl.BlockSpec((1, _P, 128), lambda i: (i, 0, 0)),
            pl.BlockSpec((1, _P, 128), lambda i: (i, 0, 0)),
            pl.BlockSpec((_R, _W), lambda i: (i, 0)),
        ],
        out_shape=(
            jax.ShapeDtypeStruct((nsteps, _P, 128), jnp.int32),
            jax.ShapeDtypeStruct((nsteps, _P, 128), jnp.int32),
            jax.ShapeDtypeStruct((N, _W), jnp.float32),
        ),
    )(xt3, maskf, codebook)

    idx_t = idx3.reshape(N)
    w_t = weighted.reshape(_NTILE, _RPT, _W)
    cb128 = jnp.pad(codebook, ((0, 0), (0, _W - D)))
    maccx = jnp.concatenate(
        [m_acc, n_acc[:, None], jnp.zeros((K, _W - D - 1), jnp.float32)],
        axis=1)
    zeros = jnp.zeros((K, _W), jnp.float32)

    mesh = plsc.VectorSubcoreMesh(core_axis_name="c", subcore_axis_name="s")
    sc = functools.partial(
        pl.kernel,
        mesh=mesh,
        out_type=(
            jax.ShapeDtypeStruct((N, _W), jnp.float32),
            jax.ShapeDtypeStruct((K, _W), jnp.float32),
        ),
        scratch_types=[
            pltpu.VMEM((_RPT,), jnp.int32),
            pltpu.VMEM((_RPT // _SUB, _SUB), jnp.int32),
            pltpu.VMEM((_CHUNK, _W), jnp.float32),
            pltpu.VMEM((_CHUNK, _W), jnp.float32),
            pltpu.VMEM((_CHUNK, _W), jnp.float32),
            pltpu.VMEM((_CHUNK, _W), jnp.float32),
            pltpu.VMEM((K // _NTILE, _W), jnp.float32),
            pltpu.VMEM((K // _NTILE, _W), jnp.float32),
            pltpu.VMEM_SHARED((K, _W), jnp.float32),
            pltpu.SemaphoreType.DMA,
            pltpu.SemaphoreType.DMA,
            pltpu.SemaphoreType.DMA,
        ],
    )(_sc_body)
    quant128, newx = sc(idx_t, w_t, cb128, maccx, zeros)

    return (quant128[:, :D].reshape(T, two, B, D).transpose(2, 0, 1, 3),
            push3.reshape(T, two, B).transpose(2, 0, 1),
            newx[:, :D],
            newx[:, D])
